# baseline (speedup 1.0000x reference)
import jax
import jax.numpy as jnp
from jax.experimental import pallas as pl
from jax.experimental.pallas import tpu as pltpu

_MARGIN = 1.0
_NUM_CLASSES = 32
_BIG = 1e9


def _dot_t(a, b):
    return jax.lax.dot_general(
        a, b, (((1,), (1,)), ((), ())), preferred_element_type=jnp.float32)


def _triplet_loss_kernel(emb_ref, lab_row_ref, out_ref):
    emb = emb_ref[...]
    n, f = emb.shape
    emb_bf = emb.astype(jnp.bfloat16)
    embm2_bf = emb_bf * jnp.bfloat16(-2.0)
    embsq = emb * emb
    ones_f = jnp.ones((1, f), jnp.float32)
    d_row = _dot_t(ones_f, embsq)
    lab_row = lab_row_ref[...]

    lvals = jax.lax.broadcasted_iota(jnp.int32, (_NUM_CLASSES, n), 0)
    onehot = (lab_row == lvals).astype(jnp.bfloat16)
    m = jnp.sum(onehot.astype(jnp.float32), axis=1, keepdims=True)
    cnt = 0.5 * (jnp.sum(m * m, keepdims=True) - jnp.float32(n))
    samef = jax.lax.dot_general(
        onehot, onehot, (((0,), (0,)), ((), ())),
        preferred_element_type=jnp.float32)

    g2 = _dot_t(emb_bf, embm2_bf)
    t = g2 + d_row
    tneg = t + jnp.float32(_BIG) * samef
    mn = jnp.min(tneg, axis=1, keepdims=True)
    cc = _MARGIN - jnp.where(mn > jnp.float32(0.5 * _BIG), t[:, 0:1], mn)
    col = jax.lax.broadcasted_iota(jnp.int32, (n, n), 1)
    row = jax.lax.broadcasted_iota(jnp.int32, (n, n), 0)
    pairmask = jnp.where(col > row, samef, 0.0)
    lm = jnp.maximum(t + cc, 0.0) * pairmask
    ones_b = jnp.ones((1, n), jnp.float32)
    colsum = jax.lax.dot_general(
        ones_b, lm, (((1,), (0,)), ((), ())),
        preferred_element_type=jnp.float32)
    loss_sum = jnp.sum(colsum, keepdims=True)
    out_ref[...] = (loss_sum / cnt).reshape(1, 1)


def kernel(embeddings, target):
    b = embeddings.shape[0]
    lab = target.astype(jnp.int32)
    out = pl.pallas_call(
        _triplet_loss_kernel,
        out_shape=jax.ShapeDtypeStruct((1, 1), jnp.float32),
    )(embeddings, lab.reshape(1, b))
    return out[0, 0]


# speedup vs baseline: 1.3030x; 1.3030x over previous
import jax
import jax.numpy as jnp
from jax.experimental import pallas as pl
from jax.experimental.pallas import tpu as pltpu

_MARGIN = 1.0
_NUM_CLASSES = 32


def _dot_t(a, b):
    return jax.lax.dot_general(
        a, b, (((1,), (1,)), ((), ())), preferred_element_type=jnp.float32)


def _triplet_loss_kernel(emb_ref, lab_row_ref, out_ref):
    emb = emb_ref[...]
    n, f = emb.shape
    emb_bf = emb.astype(jnp.bfloat16)
    embm2_bf = emb_bf * jnp.bfloat16(-2.0)
    embsq = emb * emb
    ones_f = jnp.ones((1, f), jnp.float32)
    d_row = _dot_t(ones_f, embsq)
    lab_row = lab_row_ref[...]

    col = jax.lax.broadcasted_iota(jnp.int32, (n, n), 1)
    row = jax.lax.broadcasted_iota(jnp.int32, (n, n), 0)
    g2 = _dot_t(emb_bf, embm2_bf)
    t = g2 + d_row
    lab_col = jnp.transpose(lab_row)
    same = lab_col == lab_row
    tneg = jnp.where(same, jnp.float32(jnp.inf), t)
    mn = jnp.min(tneg, axis=1, keepdims=True)
    cc = _MARGIN - jnp.where(jnp.isinf(mn), t[:, 0:1], mn)
    upper = col > row
    losses = jnp.maximum(t + cc, 0.0)
    lm = jnp.where(same & upper, losses, 0.0)
    ones_b = jnp.ones((1, n), jnp.float32)
    colsum = jax.lax.dot_general(
        ones_b, lm, (((1,), (0,)), ((), ())),
        preferred_element_type=jnp.float32)
    loss_sum = jnp.sum(colsum, keepdims=True)

    lvals = jax.lax.broadcasted_iota(jnp.int32, (_NUM_CLASSES, n), 0)
    onehot = (lab_row == lvals).astype(jnp.float32)
    m = jnp.sum(onehot, axis=1, keepdims=True)
    cnt = 0.5 * (jnp.sum(m * m, keepdims=True) - jnp.float32(n))
    out_ref[...] = (loss_sum / cnt).reshape(1, 1)


def kernel(embeddings, target):
    b = embeddings.shape[0]
    lab = target.astype(jnp.int32)
    out = pl.pallas_call(
        _triplet_loss_kernel,
        out_shape=jax.ShapeDtypeStruct((1, 1), jnp.float32),
    )(embeddings, lab.reshape(1, b))
    return out[0, 0]
